# baseline (speedup 1.0000x reference)
import functools

import jax
import jax.numpy as jnp
from jax import lax
from jax.experimental import pallas as pl
from jax.experimental.pallas import tpu as pltpu
from jax.experimental.pallas import tpu_sc as plsc

_NC = 2
_NS = 16
_NW = _NC * _NS
_BBLK = 128


@functools.partial(jax.jit, static_argnums=(6, 7, 8))
def _embed_sum(xt, pt, nt, tok, pos, npos, b, t_len, d):
    mesh = plsc.VectorSubcoreMesh(core_axis_name="c", subcore_axis_name="s",
                                  num_cores=_NC, num_subcores=_NS)
    ngrp = _BBLK // 16

    @functools.partial(
        pl.kernel,
        out_type=jax.ShapeDtypeStruct((t_len * d, b), jnp.float32),
        mesh=mesh,
        compiler_params=pltpu.CompilerParams(use_tc_tiling_on_sc=False,
                                             needs_layout_passes=False),
        scratch_types=[
            pltpu.VMEM((2, _BBLK), jnp.int32),
            pltpu.VMEM((2, _BBLK), jnp.int32),
            pltpu.VMEM((2, _BBLK), jnp.int32),
            pltpu.VMEM((2, _BBLK, d), jnp.float32),
            pltpu.VMEM((2, d, _BBLK), jnp.float32),
            pltpu.VMEM((t_len, d), jnp.float32),
            pltpu.VMEM((t_len, d), jnp.float32),
            pltpu.SemaphoreType.DMA,
            pltpu.SemaphoreType.DMA,
            pltpu.SemaphoreType.DMA,
            pltpu.SemaphoreType.DMA,
        ],
    )
    def k(xt_hbm, pt_hbm, nt_hbm, tok_hbm, pos_hbm, npos_hbm, out_hbm,
          xidx, pidx, nidx, tbuf, obuf, ptab, ntab,
          isem0, isem1, gsem0, gsem1):
        wid = lax.axis_index("s") * _NC + lax.axis_index("c")
        b0 = wid * _BBLK
        isem = (isem0, isem1)
        gsem = (gsem0, gsem1)

        pltpu.sync_copy(pos_hbm, ptab)
        pltpu.sync_copy(npos_hbm, ntab)

        def i_issue(t, s):
            pltpu.async_copy(xt_hbm.at[t, pl.ds(b0, _BBLK)], xidx.at[s],
                             isem[s])
            pltpu.async_copy(pt_hbm.at[t, pl.ds(b0, _BBLK)], pidx.at[s],
                             isem[s])
            pltpu.async_copy(nt_hbm.at[t, pl.ds(b0, _BBLK)], nidx.at[s],
                             isem[s])

        def i_wait(s):
            pltpu.make_async_copy(xt_hbm.at[0, pl.ds(b0, _BBLK)], xidx.at[s],
                                  isem[s]).wait()
            pltpu.make_async_copy(pt_hbm.at[0, pl.ds(b0, _BBLK)], pidx.at[s],
                                  isem[s]).wait()
            pltpu.make_async_copy(nt_hbm.at[0, pl.ds(b0, _BBLK)], nidx.at[s],
                                  isem[s]).wait()

        def g_issue(s):
            pltpu.async_copy(tok_hbm.at[xidx.at[s]], tbuf.at[s], gsem[s])

        def g_wait(s):
            pltpu.make_async_copy(tok_hbm.at[xidx.at[s]], tbuf.at[s],
                                  gsem[s]).wait()

        lanes = lax.iota(jnp.int32, 16)

        def transpose_sum(s):
            rowids = [g * 16 + lanes for g in range(ngrp)]
            pvs = [pidx[s, pl.ds(g * 16, 16)] for g in range(ngrp)]
            nvs = [nidx[s, pl.ds(g * 16, 16)] for g in range(ngrp)]
            tb = tbuf.at[s]

            def d_body(dd, carry):
                dvec = jnp.full((16,), dd, jnp.int32)
                for g in range(ngrp):
                    val = (plsc.load_gather(tb, [rowids[g], dvec])
                           + plsc.load_gather(ptab, [pvs[g], dvec])
                           + plsc.load_gather(ntab, [nvs[g], dvec]))
                    obuf[s, dd, pl.ds(g * 16, 16)] = val
                return carry

            lax.fori_loop(0, d, d_body, 0)

        i_issue(0, 0)
        i_wait(0)
        g_issue(0)
        i_issue(jnp.minimum(1, t_len - 1), 1)

        def half(t, s):
            so = 1 - s
            i_wait(so)
            g_issue(so)
            g_wait(s)
            transpose_sum(s)
            i_issue(jnp.minimum(t + 2, t_len - 1), s)
            pltpu.sync_copy(obuf.at[s],
                            out_hbm.at[pl.ds(t * d, d), pl.ds(b0, _BBLK)])

        def body(i, carry):
            t = i * 2
            half(t, 0)
            half(t + 1, 1)
            return carry

        lax.fori_loop(0, t_len // 2, body, 0)
        i_wait(1)
        g_wait(0)

    return k(xt, pt, nt, tok, pos, npos)


def kernel(x, positions, next_positions, token_table, pos_table, npos_table):
    b, t_len = x.shape
    d = token_table.shape[1]
    xt = jnp.transpose(x).astype(jnp.int32)
    pt = jnp.transpose(positions).astype(jnp.int32)
    nt = jnp.transpose(next_positions).astype(jnp.int32)
    out2 = _embed_sum(xt, pt, nt, token_table, pos_table, npos_table,
                      b, t_len, d)
    return jnp.transpose(out2.reshape(t_len, d, b), (2, 0, 1))


# speedup vs baseline: 1.5590x; 1.5590x over previous
import functools

import jax
import jax.numpy as jnp
from jax import lax
from jax.experimental import pallas as pl
from jax.experimental.pallas import tpu as pltpu
from jax.experimental.pallas import tpu_sc as plsc

_NC = 2
_NS = 16
_NW = _NC * _NS
_BBLK = 128
_PAD = 69


@functools.partial(jax.jit, static_argnums=(6, 7, 8))
def _embed_sum(xt, pt, nt, tok, pos, npos, b, t_len, d):
    mesh = plsc.VectorSubcoreMesh(core_axis_name="c", subcore_axis_name="s",
                                  num_cores=_NC, num_subcores=_NS)
    ngrp = _BBLK // 16
    nvec = d // 16

    @functools.partial(
        pl.kernel,
        out_type=jax.ShapeDtypeStruct((t_len * d, b), jnp.float32),
        mesh=mesh,
        compiler_params=pltpu.CompilerParams(use_tc_tiling_on_sc=False,
                                             needs_layout_passes=False),
        scratch_types=[
            pltpu.VMEM((2, _BBLK), jnp.int32),
            pltpu.VMEM((2, _BBLK), jnp.int32),
            pltpu.VMEM((2, _BBLK), jnp.int32),
            pltpu.VMEM((2, _BBLK, d), jnp.float32),
            pltpu.VMEM((2, _BBLK, _PAD), jnp.float32),
            pltpu.VMEM((2, d, _BBLK), jnp.float32),
            pltpu.VMEM((200, d), jnp.float32),
            pltpu.VMEM((200, _PAD), jnp.float32),
            pltpu.VMEM((200, _PAD), jnp.float32),
            pltpu.SemaphoreType.DMA,
            pltpu.SemaphoreType.DMA,
            pltpu.SemaphoreType.DMA,
            pltpu.SemaphoreType.DMA,
        ],
    )
    def k(xt_hbm, pt_hbm, nt_hbm, tok_hbm, pos_hbm, npos_hbm, out_hbm,
          xidx, pidx, nidx, tbuf, t69, obuf, ptmp, ptab, ntab,
          isem0, isem1, gsem0, gsem1):
        wid = lax.axis_index("s") * _NC + lax.axis_index("c")
        b0 = wid * _BBLK
        isem = (isem0, isem1)
        gsem = (gsem0, gsem1)

        def repack(src, dst, nrows):
            def rbody(r, carry):
                for j in range(nvec):
                    sl = pl.ds(j * 16, 16)
                    dst[r, sl] = src[r, sl]
                return carry
            lax.fori_loop(0, nrows, rbody, 0)

        pltpu.sync_copy(pos_hbm, ptmp)
        repack(ptmp, ptab, 200)
        pltpu.sync_copy(npos_hbm, ptmp)
        repack(ptmp, ntab, 200)

        def i_issue(t, s):
            pltpu.async_copy(xt_hbm.at[t, pl.ds(b0, _BBLK)], xidx.at[s],
                             isem[s])
            pltpu.async_copy(pt_hbm.at[t, pl.ds(b0, _BBLK)], pidx.at[s],
                             isem[s])
            pltpu.async_copy(nt_hbm.at[t, pl.ds(b0, _BBLK)], nidx.at[s],
                             isem[s])

        def i_wait(s):
            pltpu.make_async_copy(xt_hbm.at[0, pl.ds(b0, _BBLK)], xidx.at[s],
                                  isem[s]).wait()
            pltpu.make_async_copy(pt_hbm.at[0, pl.ds(b0, _BBLK)], pidx.at[s],
                                  isem[s]).wait()
            pltpu.make_async_copy(nt_hbm.at[0, pl.ds(b0, _BBLK)], nidx.at[s],
                                  isem[s]).wait()

        def g_issue(s):
            pltpu.async_copy(tok_hbm.at[xidx.at[s]], tbuf.at[s], gsem[s])

        def g_wait(s):
            pltpu.make_async_copy(tok_hbm.at[xidx.at[s]], tbuf.at[s],
                                  gsem[s]).wait()

        lanes = lax.iota(jnp.int32, 16)

        def transpose_sum(s):
            def rbody(i, carry):
                for j in range(nvec):
                    sl = pl.ds(j * 16, 16)
                    t69[s, i, sl] = tbuf[s, i, sl]
                return carry
            lax.fori_loop(0, _BBLK, rbody, 0)

            tb = t69.at[s]

            def gbody(g, carry):
                g16 = g * 16
                rowid = g16 + lanes
                pv = pidx[s, pl.ds(g16, 16)]
                nv = nidx[s, pl.ds(g16, 16)]
                for dd in range(d):
                    dvec = jnp.full((16,), dd, jnp.int32)
                    val = (plsc.load_gather(tb, [rowid, dvec])
                           + plsc.load_gather(ptab, [pv, dvec])
                           + plsc.load_gather(ntab, [nv, dvec]))
                    obuf[s, dd, pl.ds(g16, 16)] = val
                return carry

            lax.fori_loop(0, ngrp, gbody, 0)

        i_issue(0, 0)
        i_wait(0)
        g_issue(0)
        i_issue(jnp.minimum(1, t_len - 1), 1)

        def half(t, s):
            so = 1 - s
            i_wait(so)
            g_issue(so)
            g_wait(s)
            transpose_sum(s)
            i_issue(jnp.minimum(t + 2, t_len - 1), s)
            pltpu.sync_copy(obuf.at[s],
                            out_hbm.at[pl.ds(t * d, d), pl.ds(b0, _BBLK)])

        def body(i, carry):
            t = i * 2
            half(t, 0)
            half(t + 1, 1)
            return carry

        lax.fori_loop(0, t_len // 2, body, 0)
        i_wait(1)
        g_wait(0)

    return k(xt, pt, nt, tok, pos, npos)


def kernel(x, positions, next_positions, token_table, pos_table, npos_table):
    b, t_len = x.shape
    d = token_table.shape[1]
    xt = jnp.transpose(x).astype(jnp.int32)
    pt = jnp.transpose(positions).astype(jnp.int32)
    nt = jnp.transpose(next_positions).astype(jnp.int32)
    out2 = _embed_sum(xt, pt, nt, token_table, pos_table, npos_table,
                      b, t_len, d)
    return jnp.transpose(out2.reshape(t_len, d, b), (2, 0, 1))


# speedup vs baseline: 2.0266x; 1.3000x over previous
import functools

import jax
import jax.numpy as jnp
from jax import lax
from jax.experimental import pallas as pl
from jax.experimental.pallas import tpu as pltpu
from jax.experimental.pallas import tpu_sc as plsc

_NC = 2
_NS = 16
_NW = _NC * _NS
_BBLK = 128
_PAD = 69


@functools.partial(jax.jit, static_argnums=(6, 7, 8))
def _embed_sum(xt, pt, nt, tok, pos, npos, b, t_len, d):
    mesh = plsc.VectorSubcoreMesh(core_axis_name="c", subcore_axis_name="s",
                                  num_cores=_NC, num_subcores=_NS)
    ngrp = _BBLK // 16
    nvec = d // 16

    @functools.partial(
        pl.kernel,
        out_type=jax.ShapeDtypeStruct((t_len * d, b), jnp.float32),
        mesh=mesh,
        compiler_params=pltpu.CompilerParams(use_tc_tiling_on_sc=False,
                                             needs_layout_passes=False),
        scratch_types=[
            pltpu.VMEM((2, _BBLK), jnp.int32),
            pltpu.VMEM((2, _BBLK), jnp.int32),
            pltpu.VMEM((2, _BBLK), jnp.int32),
            pltpu.VMEM((2, _BBLK, d), jnp.float32),
            pltpu.VMEM((2, _BBLK, d), jnp.float32),
            pltpu.VMEM((2, _BBLK, d), jnp.float32),
            pltpu.VMEM((_BBLK * _PAD,), jnp.float32),
            pltpu.VMEM((2, d, _BBLK), jnp.float32),
            pltpu.SemaphoreType.DMA,
            pltpu.SemaphoreType.DMA,
            pltpu.SemaphoreType.DMA,
            pltpu.SemaphoreType.DMA,
        ],
    )
    def k(xt_hbm, pt_hbm, nt_hbm, tok_hbm, pos_hbm, npos_hbm, out_hbm,
          xidx, pidx, nidx, tbuf, pbuf, nbuf, t69, obuf,
          isem0, isem1, gsem0, gsem1):
        wid = lax.axis_index("s") * _NC + lax.axis_index("c")
        b0 = wid * _BBLK
        isem = (isem0, isem1)
        gsem = (gsem0, gsem1)

        def i_issue(t, s):
            pltpu.async_copy(xt_hbm.at[t, pl.ds(b0, _BBLK)], xidx.at[s],
                             isem[s])
            pltpu.async_copy(pt_hbm.at[t, pl.ds(b0, _BBLK)], pidx.at[s],
                             isem[s])
            pltpu.async_copy(nt_hbm.at[t, pl.ds(b0, _BBLK)], nidx.at[s],
                             isem[s])

        def i_wait(s):
            pltpu.make_async_copy(xt_hbm.at[0, pl.ds(b0, _BBLK)], xidx.at[s],
                                  isem[s]).wait()
            pltpu.make_async_copy(pt_hbm.at[0, pl.ds(b0, _BBLK)], pidx.at[s],
                                  isem[s]).wait()
            pltpu.make_async_copy(nt_hbm.at[0, pl.ds(b0, _BBLK)], nidx.at[s],
                                  isem[s]).wait()

        def g_issue(s):
            pltpu.async_copy(tok_hbm.at[xidx.at[s]], tbuf.at[s], gsem[s])
            pltpu.async_copy(pos_hbm.at[pidx.at[s]], pbuf.at[s], gsem[s])
            pltpu.async_copy(npos_hbm.at[nidx.at[s]], nbuf.at[s], gsem[s])

        def g_wait(s):
            pltpu.make_async_copy(tok_hbm.at[xidx.at[s]], tbuf.at[s],
                                  gsem[s]).wait()
            pltpu.make_async_copy(pos_hbm.at[pidx.at[s]], pbuf.at[s],
                                  gsem[s]).wait()
            pltpu.make_async_copy(npos_hbm.at[nidx.at[s]], nbuf.at[s],
                                  gsem[s]).wait()

        lanes = lax.iota(jnp.int32, 16)

        def compute(s):
            def rbody(i2, carry):
                for u in range(2):
                    i = i2 * 2 + u
                    for j in range(nvec):
                        sl = pl.ds(j * 16, 16)
                        t69[pl.ds(i * _PAD + j * 16, 16)] = (
                            tbuf[s, i, sl] + pbuf[s, i, sl] + nbuf[s, i, sl])
                return carry

            lax.fori_loop(0, _BBLK // 2, rbody, 0)

            def gbody(g, carry):
                row69 = (g * 16 + lanes) * _PAD
                for dd in range(d):
                    obuf[s, dd, pl.ds(g * 16, 16)] = plsc.load_gather(
                        t69, [row69 + dd])
                return carry

            lax.fori_loop(0, ngrp, gbody, 0)

        i_issue(0, 0)
        i_wait(0)
        g_issue(0)
        i_issue(jnp.minimum(1, t_len - 1), 1)

        def half(t, s):
            so = 1 - s
            i_wait(so)
            g_issue(so)
            g_wait(s)
            compute(s)
            i_issue(jnp.minimum(t + 2, t_len - 1), s)
            pltpu.sync_copy(obuf.at[s],
                            out_hbm.at[pl.ds(t * d, d), pl.ds(b0, _BBLK)])

        def body(i, carry):
            t = i * 2
            half(t, 0)
            half(t + 1, 1)
            return carry

        lax.fori_loop(0, t_len // 2, body, 0)
        i_wait(1)
        g_wait(0)

    return k(xt, pt, nt, tok, pos, npos)


def kernel(x, positions, next_positions, token_table, pos_table, npos_table):
    b, t_len = x.shape
    d = token_table.shape[1]
    xt = jnp.transpose(x).astype(jnp.int32)
    pt = jnp.transpose(positions).astype(jnp.int32)
    nt = jnp.transpose(next_positions).astype(jnp.int32)
    out2 = _embed_sum(xt, pt, nt, token_table, pos_table, npos_table,
                      b, t_len, d)
    return jnp.transpose(out2.reshape(t_len, d, b), (2, 0, 1))


# speedup vs baseline: 2.3410x; 1.1551x over previous
import functools

import jax
import jax.numpy as jnp
from jax import lax
from jax.experimental import pallas as pl
from jax.experimental.pallas import tpu as pltpu
from jax.experimental.pallas import tpu_sc as plsc

_NC = 2
_NS = 16
_NW = _NC * _NS
_BBLK = 128
_TC = 40


@functools.partial(jax.jit, static_argnums=(6, 7, 8))
def _embed_sum(x, p, n, tok, pos, npos, b, t_len, d):
    mesh = plsc.VectorSubcoreMesh(core_axis_name="c", subcore_axis_name="s",
                                  num_cores=_NC, num_subcores=_NS)
    nch = t_len // _TC
    nvec = d // 16

    @functools.partial(
        pl.kernel,
        out_type=jax.ShapeDtypeStruct((b, t_len * d), jnp.float32),
        mesh=mesh,
        compiler_params=pltpu.CompilerParams(use_tc_tiling_on_sc=False,
                                             needs_layout_passes=False),
        scratch_types=[
            pltpu.VMEM((2 * _BBLK, _TC), jnp.int32),
            pltpu.VMEM((2 * _BBLK, _TC), jnp.int32),
            pltpu.VMEM((2 * _BBLK, _TC), jnp.int32),
            pltpu.VMEM((_TC, _BBLK), jnp.int32),
            pltpu.VMEM((_TC, _BBLK), jnp.int32),
            pltpu.VMEM((_TC, _BBLK), jnp.int32),
            pltpu.VMEM((2, _BBLK, d), jnp.float32),
            pltpu.VMEM((2, _BBLK, d), jnp.float32),
            pltpu.VMEM((2, _BBLK, d), jnp.float32),
            pltpu.VMEM((2, _BBLK, d), jnp.float32),
            pltpu.SemaphoreType.DMA,
            pltpu.SemaphoreType.DMA,
            pltpu.SemaphoreType.DMA,
            pltpu.SemaphoreType.DMA,
            pltpu.SemaphoreType.DMA,
            pltpu.SemaphoreType.DMA,
        ],
    )
    def k(x_hbm, p_hbm, n_hbm, tok_hbm, pos_hbm, npos_hbm, out_hbm,
          xi, pi, ni, xs, ps, ns, tbuf, pbuf, nbuf, obuf,
          csem0, csem1, gsem0, gsem1, ssem0, ssem1):
        wid = lax.axis_index("s") * _NC + lax.axis_index("c")
        b0 = wid * _BBLK
        csem = (csem0, csem1)
        gsem = (gsem0, gsem1)
        ssem = (ssem0, ssem1)
        srcs = ((x_hbm, xi, xs), (p_hbm, pi, ps), (n_hbm, ni, ns))
        lanes = lax.iota(jnp.int32, 16)

        def cissue(c):
            sl = c % 2
            for hbm, buf, _ in srcs:
                pltpu.async_copy(
                    hbm.at[pl.ds(b0, _BBLK), pl.ds(c * _TC, _TC)],
                    buf.at[pl.ds(sl * _BBLK, _BBLK)], csem[sl])

        def cwait(c):
            sl = c % 2
            for hbm, buf, _ in srcs:
                pltpu.make_async_copy(
                    hbm.at[pl.ds(b0, _BBLK), pl.ds(0, _TC)],
                    buf.at[pl.ds(sl * _BBLK, _BBLK)], csem[sl]).wait()

        def bulk_ext(c):
            sl = c % 2

            def jbody(j, carry):
                zj = lanes * 0 + j
                for _, buf, stg in srcs:
                    for g in range(_BBLK // 16):
                        stg[j, pl.ds(g * 16, 16)] = plsc.load_gather(
                            buf, [sl * _BBLK + g * 16 + lanes, zj])
                return carry

            lax.fori_loop(0, _TC, jbody, 0)

        def g_issue(j, s):
            pltpu.async_copy(tok_hbm.at[xs.at[j]], tbuf.at[s], gsem[s])
            pltpu.async_copy(pos_hbm.at[ps.at[j]], pbuf.at[s], gsem[s])
            pltpu.async_copy(npos_hbm.at[ns.at[j]], nbuf.at[s], gsem[s])

        def g_wait(s):
            pltpu.make_async_copy(tok_hbm.at[xs.at[0]], tbuf.at[s],
                                  gsem[s]).wait()
            pltpu.make_async_copy(pos_hbm.at[ps.at[0]], pbuf.at[s],
                                  gsem[s]).wait()
            pltpu.make_async_copy(npos_hbm.at[ns.at[0]], nbuf.at[s],
                                  gsem[s]).wait()

        def s_issue(c, j, s):
            pltpu.async_copy(
                obuf.at[s],
                out_hbm.at[pl.ds(b0, _BBLK), pl.ds((c * _TC + j) * d, d)],
                ssem[s])

        def swait(s):
            pltpu.make_async_copy(
                obuf.at[s],
                out_hbm.at[pl.ds(b0, _BBLK), pl.ds(0, d)], ssem[s]).wait()

        def compute(s):
            def rbody(i2, carry):
                for u in range(2):
                    i = i2 * 2 + u
                    for v in range(nvec):
                        sv = pl.ds(v * 16, 16)
                        obuf[s, i, sv] = (tbuf[s, i, sv] + pbuf[s, i, sv]
                                          + nbuf[s, i, sv])
                return carry

            lax.fori_loop(0, _BBLK // 2, rbody, 0)

        def half(c, j, s, first):
            g_issue(jnp.minimum(j + 1, _TC - 1), 1 - s)
            g_wait(s)
            if not first:
                swait(s)
            compute(s)
            s_issue(c, j, s)

        cissue(0)
        cissue(1)
        for c in range(nch):
            cwait(c)
            bulk_ext(c)
            if c + 2 < nch:
                cissue(c + 2)
            g_issue(0, 0)
            if c == 0:
                half(0, 0, 0, True)
                half(0, 1, 1, True)
                lo = 1
            else:
                lo = 0

            def body(i, carry, c=c):
                j = i * 2
                half(c, j, 0, False)
                half(c, j + 1, 1, False)
                return carry

            lax.fori_loop(lo, _TC // 2, body, 0)
            g_wait(0)
        swait(0)
        swait(1)

    return k(x, p, n, tok, pos, npos)


def kernel(x, positions, next_positions, token_table, pos_table, npos_table):
    b, t_len = x.shape
    d = token_table.shape[1]
    out = _embed_sum(x.astype(jnp.int32), positions.astype(jnp.int32),
                     next_positions.astype(jnp.int32),
                     token_table, pos_table, npos_table, b, t_len, d)
    return out.reshape(b, t_len, d)


# speedup vs baseline: 2.3451x; 1.0017x over previous
import functools

import jax
import jax.numpy as jnp
from jax import lax
from jax.experimental import pallas as pl
from jax.experimental.pallas import tpu as pltpu
from jax.experimental.pallas import tpu_sc as plsc

_NC = 2
_NS = 16
_NW = _NC * _NS
_BBLK = 128
_TC = 40


@functools.partial(jax.jit, static_argnums=(6, 7, 8))
def _embed_sum(x, p, n, tok, pos, npos, b, t_len, d):
    mesh = plsc.VectorSubcoreMesh(core_axis_name="c", subcore_axis_name="s",
                                  num_cores=_NC, num_subcores=_NS)
    nch = t_len // _TC
    nvec = d // 16

    @functools.partial(
        pl.kernel,
        out_type=jax.ShapeDtypeStruct((b, t_len * d), jnp.float32),
        mesh=mesh,
        compiler_params=pltpu.CompilerParams(use_tc_tiling_on_sc=False,
                                             needs_layout_passes=False),
        scratch_types=[
            pltpu.VMEM((2 * _BBLK, _TC), jnp.int32),
            pltpu.VMEM((2 * _BBLK, _TC), jnp.int32),
            pltpu.VMEM((2 * _BBLK, _TC), jnp.int32),
            pltpu.VMEM((_TC, _BBLK), jnp.int32),
            pltpu.VMEM((_TC, _BBLK), jnp.int32),
            pltpu.VMEM((_TC, _BBLK), jnp.int32),
            pltpu.VMEM((2, _BBLK, d), jnp.float32),
            pltpu.VMEM((2, _BBLK, d), jnp.float32),
            pltpu.VMEM((2, _BBLK, d), jnp.float32),
            pltpu.VMEM((_BBLK, d), jnp.float32),
            pltpu.SemaphoreType.DMA,
            pltpu.SemaphoreType.DMA,
            pltpu.SemaphoreType.DMA,
            pltpu.SemaphoreType.DMA,
            pltpu.SemaphoreType.DMA,
        ],
    )
    def k(x_hbm, p_hbm, n_hbm, tok_hbm, pos_hbm, npos_hbm, out_hbm,
          xi, pi, ni, xs, ps, ns, tbuf, pbuf, nbuf, obuf,
          csem0, csem1, gsem0, gsem1, ssem):
        wid = lax.axis_index("s") * _NC + lax.axis_index("c")
        b0 = wid * _BBLK
        csem = (csem0, csem1)
        gsem = (gsem0, gsem1)
        srcs = ((x_hbm, xi, xs), (p_hbm, pi, ps), (n_hbm, ni, ns))
        lanes = lax.iota(jnp.int32, 16)

        def cissue(c):
            sl = c % 2
            for hbm, buf, _ in srcs:
                pltpu.async_copy(
                    hbm.at[pl.ds(b0, _BBLK), pl.ds(c * _TC, _TC)],
                    buf.at[pl.ds(sl * _BBLK, _BBLK)], csem[sl])

        def cwait(c):
            sl = c % 2
            for hbm, buf, _ in srcs:
                pltpu.make_async_copy(
                    hbm.at[pl.ds(b0, _BBLK), pl.ds(0, _TC)],
                    buf.at[pl.ds(sl * _BBLK, _BBLK)], csem[sl]).wait()

        def bulk_ext(c):
            sl = c % 2

            def jbody(j, carry):
                zj = lanes * 0 + j
                for _, buf, stg in srcs:
                    for g in range(_BBLK // 16):
                        stg[j, pl.ds(g * 16, 16)] = plsc.load_gather(
                            buf, [sl * _BBLK + g * 16 + lanes, zj])
                return carry

            lax.fori_loop(0, _TC, jbody, 0)

        def g_issue(j, s):
            pltpu.async_copy(tok_hbm.at[xs.at[j]], tbuf.at[s], gsem[s])
            pltpu.async_copy(pos_hbm.at[ps.at[j]], pbuf.at[s], gsem[s])
            pltpu.async_copy(npos_hbm.at[ns.at[j]], nbuf.at[s], gsem[s])

        def g_wait(s):
            pltpu.make_async_copy(tok_hbm.at[xs.at[0]], tbuf.at[s],
                                  gsem[s]).wait()
            pltpu.make_async_copy(pos_hbm.at[ps.at[0]], pbuf.at[s],
                                  gsem[s]).wait()
            pltpu.make_async_copy(npos_hbm.at[ns.at[0]], nbuf.at[s],
                                  gsem[s]).wait()

        def s_issue(c, j):
            pltpu.async_copy(
                obuf,
                out_hbm.at[pl.ds(b0, _BBLK), pl.ds((c * _TC + j) * d, d)],
                ssem)

        def swait():
            pltpu.make_async_copy(
                obuf,
                out_hbm.at[pl.ds(b0, _BBLK), pl.ds(0, d)], ssem).wait()

        def compute(s):
            def rbody(i2, carry):
                for u in range(2):
                    i = i2 * 2 + u
                    for v in range(nvec):
                        sv = pl.ds(v * 16, 16)
                        obuf[i, sv] = (tbuf[s, i, sv] + pbuf[s, i, sv]
                                       + nbuf[s, i, sv])
                return carry

            lax.fori_loop(0, _BBLK // 2, rbody, 0)

        def half(c, j, s, first):
            g_issue(jnp.minimum(j + 1, _TC - 1), 1 - s)
            g_wait(s)
            if not first:
                swait()
            compute(s)
            s_issue(c, j)

        cissue(0)
        cissue(1)
        for c in range(nch):
            cwait(c)
            bulk_ext(c)
            if c + 2 < nch:
                cissue(c + 2)
            g_issue(0, 0)
            if c == 0:
                half(0, 0, 0, True)
                half(0, 1, 1, False)
                lo = 1
            else:
                lo = 0

            def body(i, carry, c=c):
                j = i * 2
                half(c, j, 0, False)
                half(c, j + 1, 1, False)
                return carry

            lax.fori_loop(lo, _TC // 2, body, 0)
            g_wait(0)
        swait()

    return k(x, p, n, tok, pos, npos)


def kernel(x, positions, next_positions, token_table, pos_table, npos_table):
    b, t_len = x.shape
    d = token_table.shape[1]
    out = _embed_sum(x.astype(jnp.int32), positions.astype(jnp.int32),
                     next_positions.astype(jnp.int32),
                     token_table, pos_table, npos_table, b, t_len, d)
    return out.reshape(b, t_len, d)
